# baseline (speedup 1.0000x reference)
import jax
import jax.numpy as jnp
from jax import lax
from jax.experimental import pallas as pl
from jax.experimental.pallas import tpu as pltpu
from jax.experimental.pallas import tpu_sc as plsc

N = 10000
E = 320000
D = 128

NC = 2
NS = 16
EPT = E // (NC * NS)
CH = 104
NCH = 96
TAIL = EPT - CH * NCH
IR = 8
NBUF = 3
N3 = 10240
RPT = N3 // NS
FB = 80


def _mesh():
  return plsc.VectorSubcoreMesh(
      core_axis_name="c", subcore_axis_name="s", num_cores=NC, num_subcores=NS
  )


def _deg_body(col_hbm, degp_hbm, idx_r, idx_t, ones_v, zero_v, sem_i, sem_s,
              deg_sh):
  cid = lax.axis_index("c")
  sid = lax.axis_index("s")
  for i in range(8):
    ones_v[pl.ds(i * 16, 16)] = jnp.ones((16,), jnp.float32)
  for i in range(RPT // 16):
    zero_v[pl.ds(i * 16, 16)] = jnp.zeros((16,), jnp.float32)

  base = pl.multiple_of((cid * NS + sid) * EPT, 8)

  def idx_load(j, slot):
    return pltpu.make_async_copy(
        col_hbm.at[pl.ds(pl.multiple_of(base + j * CH, 8), CH)],
        idx_r.at[slot],
        sem_i,
    )

  def scat(slot):
    return pltpu.make_async_copy(
        ones_v.at[pl.ds(0, CH)], deg_sh.at[idx_r.at[slot]], sem_s
    )

  pltpu.sync_copy(zero_v, deg_sh.at[pl.ds(sid * RPT, RPT)])
  for j in range(3):
    idx_load(j, j).start()
  plsc.subcore_barrier()

  def step(j, carry):
    @pl.when(j + 3 < NCH)
    def _():
      idx_load(j + 3, lax.rem(j + 3, IR)).start()

    s = lax.rem(j, IR)
    idx_load(j, s).wait()
    pltpu.async_copy(
        ones_v.at[pl.ds(0, CH)], deg_sh.at[idx_r.at[s]], sem_s, add=True
    )

    @pl.when(j >= 1)
    def _():
      scat(lax.rem(j - 1, IR)).wait()

    return carry

  lax.fori_loop(0, NCH, step, 0)
  scat(lax.rem(NCH - 1, IR)).wait()
  pltpu.sync_copy(col_hbm.at[pl.ds(base + NCH * CH, TAIL)], idx_t)
  pltpu.sync_copy(ones_v.at[pl.ds(0, TAIL)], deg_sh.at[idx_t], add=True)
  plsc.subcore_barrier()
  pltpu.sync_copy(
      deg_sh.at[pl.ds(sid * RPT, RPT)],
      degp_hbm.at[pl.ds(cid * N3 + sid * RPT, RPT)],
  )


_deg_kernel = pl.kernel(
    _deg_body,
    out_type=jax.ShapeDtypeStruct((NC * N3,), jnp.float32),
    mesh=_mesh(),
    scratch_types=[
        pltpu.VMEM((IR, CH), jnp.int32),
        pltpu.VMEM((TAIL,), jnp.int32),
        pltpu.VMEM((128,), jnp.float32),
        pltpu.VMEM((RPT,), jnp.float32),
        pltpu.SemaphoreType.DMA,
        pltpu.SemaphoreType.DMA,
        pltpu.VMEM_SHARED((N3,), jnp.float32),
    ],
)


def _norm_body(x_ref, degc_ref, y_ref, dis_ref):
  deg = degc_ref[:, 0:1] + degc_ref[:, 1:2] + 1.0
  dis = lax.rsqrt(deg)
  dis_ref[...] = dis
  y_ref[...] = x_ref[...] * dis


def _norm(x, degc):
  return pl.pallas_call(
      _norm_body,
      out_shape=(
          jax.ShapeDtypeStruct((N3, D), jnp.float32),
          jax.ShapeDtypeStruct((N3, 1), jnp.float32),
      ),
  )(x, degc)


def _mp_body(
    row_hbm, col_hbm, y_hbm, p_hbm, idxr_r, idxc_r, idxr_t, idxc_t, rows_v,
    sem_i, sem_g, sem_s, acc_sh
):
  cid = lax.axis_index("c")
  sid = lax.axis_index("s")
  base = pl.multiple_of((cid * NS + sid) * EPT, 8)
  rbase = pl.multiple_of(sid * RPT, 8)

  def idx_load(j, slot, which):
    src = row_hbm if which == 0 else col_hbm
    dst = idxr_r if which == 0 else idxc_r
    return pltpu.make_async_copy(
        src.at[pl.ds(pl.multiple_of(base + j * CH, 8), CH)],
        dst.at[slot],
        sem_i,
    )

  def gath(slot, b):
    return pltpu.make_async_copy(
        y_hbm.at[idxr_r.at[slot]], rows_v.at[b, pl.ds(0, CH)], sem_g
    )

  def scat(slot, b):
    return pltpu.make_async_copy(
        rows_v.at[b, pl.ds(0, CH)], acc_sh.at[idxc_r.at[slot]], sem_s
    )

  init_sl = pl.ds(rbase, RPT)
  pltpu.sync_copy(y_hbm.at[init_sl], acc_sh.at[init_sl])
  for j in range(3):
    idx_load(j, j, 0).start()
    idx_load(j, j, 1).start()
  plsc.subcore_barrier()
  for j in range(2):
    idx_load(j, j, 0).wait()
    idx_load(j, j, 1).wait()
    gath(j, j).start()

  def step(j, carry):
    @pl.when(j + 3 < NCH)
    def _():
      s3 = lax.rem(j + 3, IR)
      idx_load(j + 3, s3, 0).start()
      idx_load(j + 3, s3, 1).start()

    s = lax.rem(j, IR)
    b = lax.rem(j, NBUF)
    gath(s, b).wait()
    pltpu.async_copy(
        rows_v.at[b, pl.ds(0, CH)], acc_sh.at[idxc_r.at[s]], sem_s, add=True
    )

    @pl.when(j >= 1)
    def _():
      scat(lax.rem(j - 1, IR), lax.rem(j - 1, NBUF)).wait()

    @pl.when(j + 2 < NCH)
    def _():
      s2 = lax.rem(j + 2, IR)
      idx_load(j + 2, s2, 0).wait()
      idx_load(j + 2, s2, 1).wait()
      gath(s2, lax.rem(j + 2, NBUF)).start()

    return carry

  lax.fori_loop(0, NCH, step, 0)
  scat(lax.rem(NCH - 1, IR), lax.rem(NCH - 1, NBUF)).wait()
  pltpu.sync_copy(row_hbm.at[pl.ds(base + NCH * CH, TAIL)], idxr_t)
  pltpu.sync_copy(col_hbm.at[pl.ds(base + NCH * CH, TAIL)], idxc_t)
  pltpu.async_copy(
      y_hbm.at[idxr_t], rows_v.at[0, pl.ds(0, TAIL)], sem_g
  ).wait()
  pltpu.sync_copy(rows_v.at[0, pl.ds(0, TAIL)], acc_sh.at[idxc_t], add=True)
  plsc.subcore_barrier()
  out_sl = pl.ds(rbase, RPT)
  pltpu.sync_copy(acc_sh.at[out_sl], p_hbm.at[cid, out_sl])


_mp_kernel = pl.kernel(
    _mp_body,
    out_type=jax.ShapeDtypeStruct((NC, N3, D), jnp.float32),
    mesh=_mesh(),
    scratch_types=[
        pltpu.VMEM((IR, CH), jnp.int32),
        pltpu.VMEM((IR, CH), jnp.int32),
        pltpu.VMEM((TAIL,), jnp.int32),
        pltpu.VMEM((TAIL,), jnp.int32),
        pltpu.VMEM((NBUF, CH, D), jnp.float32),
        pltpu.SemaphoreType.DMA,
        pltpu.SemaphoreType.DMA,
        pltpu.SemaphoreType.DMA,
        pltpu.VMEM_SHARED((N3, D), jnp.float32),
    ],
)


def _fin_body(p_ref, y_ref, dis_ref, out_ref):
  out_ref[...] = dis_ref[...] * (p_ref[0] + p_ref[1] - y_ref[...])


def _finalize(p, y, dis):
  return pl.pallas_call(
      _fin_body,
      grid=(N // FB,),
      in_specs=[
          pl.BlockSpec((NC, FB, D), lambda i: (0, i, 0)),
          pl.BlockSpec((FB, D), lambda i: (i, 0)),
          pl.BlockSpec((FB, 1), lambda i: (i, 0)),
      ],
      out_specs=pl.BlockSpec((FB, D), lambda i: (i, 0)),
      out_shape=jax.ShapeDtypeStruct((N, D), jnp.float32),
  )(p, y, dis)


def kernel(x, edge_index):
  row = edge_index[0]
  col = edge_index[1]
  x_pad = jnp.pad(x, ((0, N3 - N), (0, 0)))
  degp = _deg_kernel(col)
  degc = degp.reshape(NC, N3).T
  y, dis = _norm(x_pad, degc)
  p = _mp_kernel(row, col, y)
  return _finalize(p, y, dis)


# speedup vs baseline: 1.3191x; 1.3191x over previous
import jax
import jax.numpy as jnp
from jax import lax
from jax.experimental import pallas as pl
from jax.experimental.pallas import tpu as pltpu
from jax.experimental.pallas import tpu_sc as plsc

N = 10000
E = 320000
D = 128

NC = 2
NS = 16
EPT = E // (NC * NS)
CH = 104
NCH = 96
TAIL = EPT - CH * NCH
IR = 8
NBUF = 3
N3 = 10240
RPT = N3 // NS
FB = 80


def _mesh():
  return plsc.VectorSubcoreMesh(
      core_axis_name="c", subcore_axis_name="s", num_cores=NC, num_subcores=NS
  )


def _deg_body(col_hbm, degp_hbm, idx_r, idx_t, ones_v, zero_v, sem_i, sem_s,
              deg_sh):
  cid = lax.axis_index("c")
  sid = lax.axis_index("s")
  for i in range(8):
    ones_v[pl.ds(i * 16, 16)] = jnp.ones((16,), jnp.float32)
  for i in range(RPT // 16):
    zero_v[pl.ds(i * 16, 16)] = jnp.zeros((16,), jnp.float32)

  base = pl.multiple_of((cid * NS + sid) * EPT, 8)

  def idx_load(j, slot):
    return pltpu.make_async_copy(
        col_hbm.at[pl.ds(pl.multiple_of(base + j * CH, 8), CH)],
        idx_r.at[slot],
        sem_i,
    )

  def scat(slot):
    return pltpu.make_async_copy(
        ones_v.at[pl.ds(0, CH)], deg_sh.at[idx_r.at[slot]], sem_s
    )

  pltpu.sync_copy(zero_v, deg_sh.at[pl.ds(sid * RPT, RPT)])
  for j in range(3):
    idx_load(j, j).start()
  plsc.subcore_barrier()

  def step(j, carry):
    @pl.when(j + 3 < NCH)
    def _():
      idx_load(j + 3, lax.rem(j + 3, IR)).start()

    s = lax.rem(j, IR)
    idx_load(j, s).wait()
    pltpu.async_copy(
        ones_v.at[pl.ds(0, CH)], deg_sh.at[idx_r.at[s]], sem_s, add=True
    )

    @pl.when(j >= 1)
    def _():
      scat(lax.rem(j - 1, IR)).wait()

    return carry

  lax.fori_loop(0, NCH, step, 0)
  scat(lax.rem(NCH - 1, IR)).wait()
  pltpu.sync_copy(col_hbm.at[pl.ds(base + NCH * CH, TAIL)], idx_t)
  pltpu.sync_copy(ones_v.at[pl.ds(0, TAIL)], deg_sh.at[idx_t], add=True)
  plsc.subcore_barrier()
  pltpu.sync_copy(
      deg_sh.at[pl.ds(sid * RPT, RPT)],
      degp_hbm.at[pl.ds(cid * N3 + sid * RPT, RPT)],
  )


_deg_kernel = pl.kernel(
    _deg_body,
    out_type=jax.ShapeDtypeStruct((NC * N3,), jnp.float32),
    mesh=_mesh(),
    scratch_types=[
        pltpu.VMEM((IR, CH), jnp.int32),
        pltpu.VMEM((TAIL,), jnp.int32),
        pltpu.VMEM((128,), jnp.float32),
        pltpu.VMEM((RPT,), jnp.float32),
        pltpu.SemaphoreType.DMA,
        pltpu.SemaphoreType.DMA,
        pltpu.VMEM_SHARED((N3,), jnp.float32),
    ],
)


def _norm_body(x_ref, degc_ref, y_ref, dis_ref):
  deg = degc_ref[:, 0:1] + degc_ref[:, 1:2] + 1.0
  dis = lax.rsqrt(deg)
  dis_ref[...] = dis
  y_ref[...] = x_ref[...] * dis


def _norm(x, degc):
  return pl.pallas_call(
      _norm_body,
      out_shape=(
          jax.ShapeDtypeStruct((N3, D), jnp.float32),
          jax.ShapeDtypeStruct((N3, 1), jnp.float32),
      ),
  )(x, degc)


def _mp_body(
    row_hbm, col_hbm, y_hbm, p_hbm, idxr_r, idxc_r, idxr_t, idxc_t, rows_v,
    sem_i, sem_g, sem_s, acc_sh
):
  cid = lax.axis_index("c")
  sid = lax.axis_index("s")
  base = pl.multiple_of((cid * NS + sid) * EPT, 8)
  rbase = pl.multiple_of(sid * RPT, 8)

  def idx_load(j, slot, which):
    src = row_hbm if which == 0 else col_hbm
    dst = idxr_r if which == 0 else idxc_r
    return pltpu.make_async_copy(
        src.at[pl.ds(pl.multiple_of(base + j * CH, 8), CH)],
        dst.at[slot],
        sem_i,
    )

  def gath(slot, b):
    return pltpu.make_async_copy(
        y_hbm.at[idxr_r.at[slot]], rows_v.at[b, pl.ds(0, CH)], sem_g
    )

  def scat(slot, b):
    return pltpu.make_async_copy(
        rows_v.at[b, pl.ds(0, CH)], acc_sh.at[idxc_r.at[slot]], sem_s
    )

  init_sl = pl.ds(rbase, RPT)
  pltpu.sync_copy(y_hbm.at[init_sl], acc_sh.at[init_sl])
  for j in range(3):
    idx_load(j, j, 0).start()
    idx_load(j, j, 1).start()
  plsc.subcore_barrier()
  for j in range(2):
    idx_load(j, j, 0).wait()
    idx_load(j, j, 1).wait()
    gath(j, j).start()

  def step(j, carry):
    @pl.when(j + 3 < NCH)
    def _():
      s3 = lax.rem(j + 3, IR)
      idx_load(j + 3, s3, 0).start()
      idx_load(j + 3, s3, 1).start()

    s = lax.rem(j, IR)
    b = lax.rem(j, NBUF)
    gath(s, b).wait()
    pltpu.async_copy(
        rows_v.at[b, pl.ds(0, CH)], acc_sh.at[idxc_r.at[s]], sem_s, add=True
    )

    @pl.when(j >= 1)
    def _():
      scat(lax.rem(j - 1, IR), lax.rem(j - 1, NBUF)).wait()

    @pl.when(j + 2 < NCH)
    def _():
      s2 = lax.rem(j + 2, IR)
      idx_load(j + 2, s2, 0).wait()
      idx_load(j + 2, s2, 1).wait()
      gath(s2, lax.rem(j + 2, NBUF)).start()

    return carry

  lax.fori_loop(0, NCH, step, 0)
  scat(lax.rem(NCH - 1, IR), lax.rem(NCH - 1, NBUF)).wait()
  pltpu.sync_copy(row_hbm.at[pl.ds(base + NCH * CH, TAIL)], idxr_t)
  pltpu.sync_copy(col_hbm.at[pl.ds(base + NCH * CH, TAIL)], idxc_t)
  pltpu.async_copy(
      y_hbm.at[idxr_t], rows_v.at[0, pl.ds(0, TAIL)], sem_g
  ).wait()
  pltpu.sync_copy(rows_v.at[0, pl.ds(0, TAIL)], acc_sh.at[idxc_t], add=True)
  plsc.subcore_barrier()
  out_sl = pl.ds(rbase, RPT)
  pltpu.sync_copy(acc_sh.at[out_sl], p_hbm.at[cid, out_sl])


_mp_kernel = pl.kernel(
    _mp_body,
    out_type=jax.ShapeDtypeStruct((NC, N3, D), jnp.float32),
    mesh=_mesh(),
    scratch_types=[
        pltpu.VMEM((IR, CH), jnp.int32),
        pltpu.VMEM((IR, CH), jnp.int32),
        pltpu.VMEM((TAIL,), jnp.int32),
        pltpu.VMEM((TAIL,), jnp.int32),
        pltpu.VMEM((NBUF, CH, D), jnp.float32),
        pltpu.SemaphoreType.DMA,
        pltpu.SemaphoreType.DMA,
        pltpu.SemaphoreType.DMA,
        pltpu.VMEM_SHARED((N3, D), jnp.float32),
    ],
)


def _fin_body(p_ref, y_ref, dis_ref, out_ref):
  out_ref[...] = dis_ref[...] * (p_ref[0] + p_ref[1] - y_ref[...])


def _finalize(p, y, dis):
  return pl.pallas_call(
      _fin_body,
      out_shape=jax.ShapeDtypeStruct((N3, D), jnp.float32),
  )(p, y, dis)


def kernel(x, edge_index):
  row = edge_index[0]
  col = edge_index[1]
  x_pad = jnp.pad(x, ((0, N3 - N), (0, 0)))
  degp = _deg_kernel(col)
  degc = degp.reshape(NC, N3).T
  y, dis = _norm(x_pad, degc)
  p = _mp_kernel(row, col, y)
  return _finalize(p, y, dis)[:N]


# speedup vs baseline: 1.3302x; 1.0083x over previous
import jax
import jax.numpy as jnp
from jax import lax
from jax.experimental import pallas as pl
from jax.experimental.pallas import tpu as pltpu
from jax.experimental.pallas import tpu_sc as plsc

N = 10000
E = 320000
D = 128

NC = 2
NS = 16
EPT = E // (NC * NS)
CH = 104
NCH = 96
TAIL = EPT - CH * NCH
IR = 8
NBUF = 3
N3 = 10240
RPT = N3 // NS
FB = 80


def _mesh():
  return plsc.VectorSubcoreMesh(
      core_axis_name="c", subcore_axis_name="s", num_cores=NC, num_subcores=NS
  )


def _deg_body(col_hbm, degp_hbm, idx_r, idx_t, ones_v, zero_v, sem_i, sem_s,
              deg_sh):
  cid = lax.axis_index("c")
  sid = lax.axis_index("s")
  for i in range(8):
    ones_v[pl.ds(i * 16, 16)] = jnp.ones((16,), jnp.float32)
  for i in range(RPT // 16):
    zero_v[pl.ds(i * 16, 16)] = jnp.zeros((16,), jnp.float32)

  base = pl.multiple_of((cid * NS + sid) * EPT, 8)

  def idx_load(j, slot):
    return pltpu.make_async_copy(
        col_hbm.at[pl.ds(pl.multiple_of(base + j * CH, 8), CH)],
        idx_r.at[slot],
        sem_i,
    )

  def scat(slot):
    return pltpu.make_async_copy(
        ones_v.at[pl.ds(0, CH)], deg_sh.at[idx_r.at[slot]], sem_s
    )

  pltpu.sync_copy(zero_v, deg_sh.at[pl.ds(sid * RPT, RPT)])
  for j in range(3):
    idx_load(j, j).start()
  plsc.subcore_barrier()

  def step(j, carry):
    @pl.when(j + 3 < NCH)
    def _():
      idx_load(j + 3, lax.rem(j + 3, IR)).start()

    s = lax.rem(j, IR)
    idx_load(j, s).wait()
    pltpu.async_copy(
        ones_v.at[pl.ds(0, CH)], deg_sh.at[idx_r.at[s]], sem_s, add=True
    )

    @pl.when(j >= 1)
    def _():
      scat(lax.rem(j - 1, IR)).wait()

    return carry

  lax.fori_loop(0, NCH, step, 0)
  scat(lax.rem(NCH - 1, IR)).wait()
  pltpu.sync_copy(col_hbm.at[pl.ds(base + NCH * CH, TAIL)], idx_t)
  pltpu.sync_copy(ones_v.at[pl.ds(0, TAIL)], deg_sh.at[idx_t], add=True)
  plsc.subcore_barrier()
  pltpu.sync_copy(
      deg_sh.at[pl.ds(sid * RPT, RPT)],
      degp_hbm.at[pl.ds(cid * N3 + sid * RPT, RPT)],
  )


_deg_kernel = pl.kernel(
    _deg_body,
    out_type=jax.ShapeDtypeStruct((NC * N3,), jnp.float32),
    mesh=_mesh(),
    scratch_types=[
        pltpu.VMEM((IR, CH), jnp.int32),
        pltpu.VMEM((TAIL,), jnp.int32),
        pltpu.VMEM((128,), jnp.float32),
        pltpu.VMEM((RPT,), jnp.float32),
        pltpu.SemaphoreType.DMA,
        pltpu.SemaphoreType.DMA,
        pltpu.VMEM_SHARED((N3,), jnp.float32),
    ],
)


def _norm_body(x_ref, degc_ref, y_ref, dis_ref):
  deg = degc_ref[:, 0:1] + degc_ref[:, 1:2] + 1.0
  dis = lax.rsqrt(deg)
  dis_ref[...] = dis
  y_ref[...] = x_ref[...] * dis


def _norm(x, degc):
  nb = N3 // 1280
  return pl.pallas_call(
      _norm_body,
      grid=(nb,),
      in_specs=[
          pl.BlockSpec((1280, D), lambda i: (i, 0)),
          pl.BlockSpec((1280, 2), lambda i: (i, 0)),
      ],
      out_specs=(
          pl.BlockSpec((1280, D), lambda i: (i, 0)),
          pl.BlockSpec((1280, 1), lambda i: (i, 0)),
      ),
      out_shape=(
          jax.ShapeDtypeStruct((N3, D), jnp.float32),
          jax.ShapeDtypeStruct((N3, 1), jnp.float32),
      ),
  )(x, degc)


def _mp_body(
    row_hbm, col_hbm, y_hbm, p_hbm, idxr_r, idxc_r, idxr_t, idxc_t, rows_v,
    sem_i, sem_g, sem_s, acc_sh
):
  cid = lax.axis_index("c")
  sid = lax.axis_index("s")
  base = pl.multiple_of((cid * NS + sid) * EPT, 8)
  rbase = pl.multiple_of(sid * RPT, 8)

  def idx_load(j, slot, which):
    src = row_hbm if which == 0 else col_hbm
    dst = idxr_r if which == 0 else idxc_r
    return pltpu.make_async_copy(
        src.at[pl.ds(pl.multiple_of(base + j * CH, 8), CH)],
        dst.at[slot],
        sem_i,
    )

  def gath(slot, b):
    return pltpu.make_async_copy(
        y_hbm.at[idxr_r.at[slot]], rows_v.at[b, pl.ds(0, CH)], sem_g
    )

  def scat(slot, b):
    return pltpu.make_async_copy(
        rows_v.at[b, pl.ds(0, CH)], acc_sh.at[idxc_r.at[slot]], sem_s
    )

  init_sl = pl.ds(rbase, RPT)
  pltpu.sync_copy(y_hbm.at[init_sl], acc_sh.at[init_sl])
  for j in range(3):
    idx_load(j, j, 0).start()
    idx_load(j, j, 1).start()
  plsc.subcore_barrier()
  for j in range(2):
    idx_load(j, j, 0).wait()
    idx_load(j, j, 1).wait()
    gath(j, j).start()

  def step(j, carry):
    @pl.when(j + 3 < NCH)
    def _():
      s3 = lax.rem(j + 3, IR)
      idx_load(j + 3, s3, 0).start()
      idx_load(j + 3, s3, 1).start()

    s = lax.rem(j, IR)
    b = lax.rem(j, NBUF)
    gath(s, b).wait()
    pltpu.async_copy(
        rows_v.at[b, pl.ds(0, CH)], acc_sh.at[idxc_r.at[s]], sem_s, add=True
    )

    @pl.when(j >= 1)
    def _():
      scat(lax.rem(j - 1, IR), lax.rem(j - 1, NBUF)).wait()

    @pl.when(j + 2 < NCH)
    def _():
      s2 = lax.rem(j + 2, IR)
      idx_load(j + 2, s2, 0).wait()
      idx_load(j + 2, s2, 1).wait()
      gath(s2, lax.rem(j + 2, NBUF)).start()

    return carry

  lax.fori_loop(0, NCH, step, 0)
  scat(lax.rem(NCH - 1, IR), lax.rem(NCH - 1, NBUF)).wait()
  pltpu.sync_copy(row_hbm.at[pl.ds(base + NCH * CH, TAIL)], idxr_t)
  pltpu.sync_copy(col_hbm.at[pl.ds(base + NCH * CH, TAIL)], idxc_t)
  pltpu.async_copy(
      y_hbm.at[idxr_t], rows_v.at[0, pl.ds(0, TAIL)], sem_g
  ).wait()
  pltpu.sync_copy(rows_v.at[0, pl.ds(0, TAIL)], acc_sh.at[idxc_t], add=True)
  plsc.subcore_barrier()
  out_sl = pl.ds(rbase, RPT)
  pltpu.sync_copy(acc_sh.at[out_sl], p_hbm.at[cid, out_sl])


_mp_kernel = pl.kernel(
    _mp_body,
    out_type=jax.ShapeDtypeStruct((NC, N3, D), jnp.float32),
    mesh=_mesh(),
    scratch_types=[
        pltpu.VMEM((IR, CH), jnp.int32),
        pltpu.VMEM((IR, CH), jnp.int32),
        pltpu.VMEM((TAIL,), jnp.int32),
        pltpu.VMEM((TAIL,), jnp.int32),
        pltpu.VMEM((NBUF, CH, D), jnp.float32),
        pltpu.SemaphoreType.DMA,
        pltpu.SemaphoreType.DMA,
        pltpu.SemaphoreType.DMA,
        pltpu.VMEM_SHARED((N3, D), jnp.float32),
    ],
)


def _fin_body(p_ref, y_ref, dis_ref, out_ref):
  out_ref[...] = dis_ref[...] * (p_ref[0] + p_ref[1] - y_ref[...])


def _finalize(p, y, dis):
  fb = 1000
  return pl.pallas_call(
      _fin_body,
      grid=(N // fb,),
      in_specs=[
          pl.BlockSpec((NC, fb, D), lambda i: (0, i, 0)),
          pl.BlockSpec((fb, D), lambda i: (i, 0)),
          pl.BlockSpec((fb, 1), lambda i: (i, 0)),
      ],
      out_specs=pl.BlockSpec((fb, D), lambda i: (i, 0)),
      out_shape=jax.ShapeDtypeStruct((N, D), jnp.float32),
  )(p, y, dis)


def kernel(x, edge_index):
  col = edge_index[1]
  row = lax.optimization_barrier((edge_index[0], col))[0]
  x_pad = jnp.pad(x, ((0, N3 - N), (0, 0)))
  degp = _deg_kernel(col)
  degc = degp.reshape(NC, N3).T
  y, dis = _norm(x_pad, degc)
  p = _mp_kernel(row, col, y)
  return _finalize(p, y, dis)


# speedup vs baseline: 1.3436x; 1.0101x over previous
import jax
import jax.numpy as jnp
from jax import lax
from jax.experimental import pallas as pl
from jax.experimental.pallas import tpu as pltpu
from jax.experimental.pallas import tpu_sc as plsc

N = 10000
E = 320000
D = 128

NC = 2
NS = 16
EPT = E // (NC * NS)
CH = 104
NCH = 96
TAIL = EPT - CH * NCH
IR = 8
NBUF = 3
N3 = 10240
RPT = N3 // NS
FB = 80


def _mesh():
  return plsc.VectorSubcoreMesh(
      core_axis_name="c", subcore_axis_name="s", num_cores=NC, num_subcores=NS
  )


def _deg_body(col_hbm, degp_hbm, idx_r, idx_t, ones_v, zero_v, sem_i, sem_s,
              deg_sh):
  cid = lax.axis_index("c")
  sid = lax.axis_index("s")
  for i in range(8):
    ones_v[pl.ds(i * 16, 16)] = jnp.ones((16,), jnp.float32)
  for i in range(RPT // 16):
    zero_v[pl.ds(i * 16, 16)] = jnp.zeros((16,), jnp.float32)

  base = pl.multiple_of((cid * NS + sid) * EPT, 8)

  def idx_load(j, slot):
    return pltpu.make_async_copy(
        col_hbm.at[pl.ds(pl.multiple_of(base + j * CH, 8), CH)],
        idx_r.at[slot],
        sem_i,
    )

  def scat(slot):
    return pltpu.make_async_copy(
        ones_v.at[pl.ds(0, CH)], deg_sh.at[idx_r.at[slot]], sem_s
    )

  pltpu.sync_copy(zero_v, deg_sh.at[pl.ds(sid * RPT, RPT)])
  for j in range(3):
    idx_load(j, j).start()
  plsc.subcore_barrier()

  def step(j, carry):
    @pl.when(j + 3 < NCH)
    def _():
      idx_load(j + 3, lax.rem(j + 3, IR)).start()

    s = lax.rem(j, IR)
    idx_load(j, s).wait()
    pltpu.async_copy(
        ones_v.at[pl.ds(0, CH)], deg_sh.at[idx_r.at[s]], sem_s, add=True
    )

    @pl.when(j >= 2)
    def _():
      scat(lax.rem(j - 2, IR)).wait()

    return carry

  lax.fori_loop(0, NCH, step, 0)
  scat(lax.rem(NCH - 2, IR)).wait()
  scat(lax.rem(NCH - 1, IR)).wait()
  pltpu.sync_copy(col_hbm.at[pl.ds(base + NCH * CH, TAIL)], idx_t)
  pltpu.sync_copy(ones_v.at[pl.ds(0, TAIL)], deg_sh.at[idx_t], add=True)
  plsc.subcore_barrier()
  pltpu.sync_copy(
      deg_sh.at[pl.ds(sid * RPT, RPT)],
      degp_hbm.at[pl.ds(cid * N3 + sid * RPT, RPT)],
  )


_deg_kernel = pl.kernel(
    _deg_body,
    out_type=jax.ShapeDtypeStruct((NC * N3,), jnp.float32),
    mesh=_mesh(),
    scratch_types=[
        pltpu.VMEM((IR, CH), jnp.int32),
        pltpu.VMEM((TAIL,), jnp.int32),
        pltpu.VMEM((128,), jnp.float32),
        pltpu.VMEM((RPT,), jnp.float32),
        pltpu.SemaphoreType.DMA,
        pltpu.SemaphoreType.DMA,
        pltpu.VMEM_SHARED((N3,), jnp.float32),
    ],
)


def _norm_body(x_ref, degc_ref, y_ref, dis_ref):
  deg = degc_ref[:, 0:1] + degc_ref[:, 1:2] + 1.0
  dis = lax.rsqrt(deg)
  dis_ref[...] = dis
  y_ref[...] = x_ref[...] * dis


def _norm(x, degc):
  return pl.pallas_call(
      _norm_body,
      out_shape=(
          jax.ShapeDtypeStruct((N3, D), jnp.float32),
          jax.ShapeDtypeStruct((N3, 1), jnp.float32),
      ),
  )(x, degc)


def _mp_body(
    row_hbm, col_hbm, y_hbm, p_hbm, idxr_r, idxc_r, idxr_t, idxc_t, rows_v,
    sem_i, sem_g, sem_s, acc_sh
):
  cid = lax.axis_index("c")
  sid = lax.axis_index("s")
  base = pl.multiple_of((cid * NS + sid) * EPT, 8)
  rbase = pl.multiple_of(sid * RPT, 8)

  def idx_load(j, slot, which):
    src = row_hbm if which == 0 else col_hbm
    dst = idxr_r if which == 0 else idxc_r
    return pltpu.make_async_copy(
        src.at[pl.ds(pl.multiple_of(base + j * CH, 8), CH)],
        dst.at[slot],
        sem_i,
    )

  def gath(slot, b):
    return pltpu.make_async_copy(
        y_hbm.at[idxr_r.at[slot]], rows_v.at[b, pl.ds(0, CH)], sem_g
    )

  def scat(slot, b):
    return pltpu.make_async_copy(
        rows_v.at[b, pl.ds(0, CH)], acc_sh.at[idxc_r.at[slot]], sem_s
    )

  init_sl = pl.ds(rbase, RPT)
  pltpu.sync_copy(y_hbm.at[init_sl], acc_sh.at[init_sl])
  for j in range(3):
    idx_load(j, j, 0).start()
    idx_load(j, j, 1).start()
  plsc.subcore_barrier()
  for j in range(2):
    idx_load(j, j, 0).wait()
    idx_load(j, j, 1).wait()
    gath(j, j).start()

  def step(j, carry):
    @pl.when(j + 3 < NCH)
    def _():
      s3 = lax.rem(j + 3, IR)
      idx_load(j + 3, s3, 0).start()
      idx_load(j + 3, s3, 1).start()

    s = lax.rem(j, IR)
    b = lax.rem(j, NBUF)
    gath(s, b).wait()
    pltpu.async_copy(
        rows_v.at[b, pl.ds(0, CH)], acc_sh.at[idxc_r.at[s]], sem_s, add=True
    )

    @pl.when(j >= 1)
    def _():
      scat(lax.rem(j - 1, IR), lax.rem(j - 1, NBUF)).wait()

    @pl.when(j + 2 < NCH)
    def _():
      s2 = lax.rem(j + 2, IR)
      idx_load(j + 2, s2, 0).wait()
      idx_load(j + 2, s2, 1).wait()
      gath(s2, lax.rem(j + 2, NBUF)).start()

    return carry

  lax.fori_loop(0, NCH, step, 0)
  scat(lax.rem(NCH - 1, IR), lax.rem(NCH - 1, NBUF)).wait()
  pltpu.sync_copy(row_hbm.at[pl.ds(base + NCH * CH, TAIL)], idxr_t)
  pltpu.sync_copy(col_hbm.at[pl.ds(base + NCH * CH, TAIL)], idxc_t)
  pltpu.async_copy(
      y_hbm.at[idxr_t], rows_v.at[0, pl.ds(0, TAIL)], sem_g
  ).wait()
  pltpu.sync_copy(rows_v.at[0, pl.ds(0, TAIL)], acc_sh.at[idxc_t], add=True)
  plsc.subcore_barrier()
  out_sl = pl.ds(rbase, RPT)
  pltpu.sync_copy(acc_sh.at[out_sl], p_hbm.at[cid, out_sl])


_mp_kernel = pl.kernel(
    _mp_body,
    out_type=jax.ShapeDtypeStruct((NC, N3, D), jnp.float32),
    mesh=_mesh(),
    scratch_types=[
        pltpu.VMEM((IR, CH), jnp.int32),
        pltpu.VMEM((IR, CH), jnp.int32),
        pltpu.VMEM((TAIL,), jnp.int32),
        pltpu.VMEM((TAIL,), jnp.int32),
        pltpu.VMEM((NBUF, CH, D), jnp.float32),
        pltpu.SemaphoreType.DMA,
        pltpu.SemaphoreType.DMA,
        pltpu.SemaphoreType.DMA,
        pltpu.VMEM_SHARED((N3, D), jnp.float32),
    ],
)


def _fin_body(p_ref, y_ref, dis_ref, out_ref):
  out_ref[...] = dis_ref[...] * (p_ref[0] + p_ref[1] - y_ref[...])


def _finalize(p, y, dis):
  fb = 1000
  return pl.pallas_call(
      _fin_body,
      grid=(N // fb,),
      in_specs=[
          pl.BlockSpec((NC, fb, D), lambda i: (0, i, 0)),
          pl.BlockSpec((fb, D), lambda i: (i, 0)),
          pl.BlockSpec((fb, 1), lambda i: (i, 0)),
      ],
      out_specs=pl.BlockSpec((fb, D), lambda i: (i, 0)),
      out_shape=jax.ShapeDtypeStruct((N, D), jnp.float32),
  )(p, y, dis)


def kernel(x, edge_index):
  col = edge_index[1]
  row = lax.optimization_barrier((edge_index[0], col))[0]
  x_pad = jnp.pad(x, ((0, N3 - N), (0, 0)))
  degp = _deg_kernel(col)
  degc = degp.reshape(NC, N3).T
  y, dis = _norm(x_pad, degc)
  p = _mp_kernel(row, col, y)
  return _finalize(p, y, dis)


# speedup vs baseline: 1.3458x; 1.0016x over previous
import jax
import jax.numpy as jnp
from jax import lax
from jax.experimental import pallas as pl
from jax.experimental.pallas import tpu as pltpu
from jax.experimental.pallas import tpu_sc as plsc

N = 10000
E = 320000
D = 128

NC = 2
NS = 16
EPT = E // (NC * NS)
CH = 104
NCH = 96
TAIL = EPT - CH * NCH
IR = 8
NBUF = 3
N3 = 10240
RPT = N3 // NS


def _mesh():
  return plsc.VectorSubcoreMesh(
      core_axis_name="c", subcore_axis_name="s", num_cores=NC, num_subcores=NS
  )


def _deg_body(col_hbm, degp_hbm, idx_r, idx_t, ones_v, zero_v, sem_i, sem_s,
              deg_sh):
  cid = lax.axis_index("c")
  sid = lax.axis_index("s")
  for i in range(8):
    ones_v[pl.ds(i * 16, 16)] = jnp.ones((16,), jnp.float32)
  for i in range(RPT // 16):
    zero_v[pl.ds(i * 16, 16)] = jnp.zeros((16,), jnp.float32)

  base = pl.multiple_of((cid * NS + sid) * EPT, 8)

  def idx_load(j, slot):
    return pltpu.make_async_copy(
        col_hbm.at[pl.ds(pl.multiple_of(base + j * CH, 8), CH)],
        idx_r.at[slot],
        sem_i,
    )

  def scat(slot):
    return pltpu.make_async_copy(
        ones_v.at[pl.ds(0, CH)], deg_sh.at[idx_r.at[slot]], sem_s
    )

  pltpu.sync_copy(zero_v, deg_sh.at[pl.ds(sid * RPT, RPT)])
  for j in range(3):
    idx_load(j, j).start()
  plsc.subcore_barrier()

  def step(j, carry):
    @pl.when(j + 3 < NCH)
    def _():
      idx_load(j + 3, lax.rem(j + 3, IR)).start()

    s = lax.rem(j, IR)
    idx_load(j, s).wait()
    pltpu.async_copy(
        ones_v.at[pl.ds(0, CH)], deg_sh.at[idx_r.at[s]], sem_s, add=True
    )

    @pl.when(j >= 2)
    def _():
      scat(lax.rem(j - 2, IR)).wait()

    return carry

  lax.fori_loop(0, NCH, step, 0)
  scat(lax.rem(NCH - 2, IR)).wait()
  scat(lax.rem(NCH - 1, IR)).wait()
  pltpu.sync_copy(col_hbm.at[pl.ds(base + NCH * CH, TAIL)], idx_t)
  pltpu.sync_copy(ones_v.at[pl.ds(0, TAIL)], deg_sh.at[idx_t], add=True)
  plsc.subcore_barrier()
  pltpu.sync_copy(
      deg_sh.at[pl.ds(sid * RPT, RPT)],
      degp_hbm.at[pl.ds(cid * N3 + sid * RPT, RPT)],
  )


_deg_kernel = pl.kernel(
    _deg_body,
    out_type=jax.ShapeDtypeStruct((NC * N3,), jnp.float32),
    mesh=_mesh(),
    scratch_types=[
        pltpu.VMEM((IR, CH), jnp.int32),
        pltpu.VMEM((TAIL,), jnp.int32),
        pltpu.VMEM((128,), jnp.float32),
        pltpu.VMEM((RPT,), jnp.float32),
        pltpu.SemaphoreType.DMA,
        pltpu.SemaphoreType.DMA,
        pltpu.VMEM_SHARED((N3,), jnp.float32),
    ],
)


def _norm_body(x_ref, degc_ref, y_ref, dis_ref):
  deg = degc_ref[:, 0:1] + degc_ref[:, 1:2] + 1.0
  dis = lax.rsqrt(deg)
  dis_ref[...] = dis
  y_ref[...] = x_ref[...] * dis


def _norm(x, degc):
  return pl.pallas_call(
      _norm_body,
      out_shape=(
          jax.ShapeDtypeStruct((N3, D), jnp.float32),
          jax.ShapeDtypeStruct((N3, 1), jnp.float32),
      ),
  )(x, degc)


def _mp_body(
    row_hbm, col_hbm, y_hbm, p_hbm, idxr_r, idxc_r, idxr_t, idxc_t, rows_v,
    sem_i, sem_g, sem_s, acc_sh
):
  cid = lax.axis_index("c")
  sid = lax.axis_index("s")
  base = pl.multiple_of((cid * NS + sid) * EPT, 8)
  rbase = pl.multiple_of(sid * RPT, 8)

  def idx_load(j, slot, which):
    src = row_hbm if which == 0 else col_hbm
    dst = idxr_r if which == 0 else idxc_r
    return pltpu.make_async_copy(
        src.at[pl.ds(pl.multiple_of(base + j * CH, 8), CH)],
        dst.at[slot],
        sem_i,
    )

  def gath(slot, b):
    return pltpu.make_async_copy(
        y_hbm.at[idxr_r.at[slot]], rows_v.at[b, pl.ds(0, CH)], sem_g
    )

  def scat(slot, b):
    return pltpu.make_async_copy(
        rows_v.at[b, pl.ds(0, CH)], acc_sh.at[idxc_r.at[slot]], sem_s
    )

  init_sl = pl.ds(rbase, RPT)
  pltpu.sync_copy(y_hbm.at[init_sl], acc_sh.at[init_sl])
  for j in range(3):
    idx_load(j, j, 0).start()
    idx_load(j, j, 1).start()
  plsc.subcore_barrier()
  for j in range(2):
    idx_load(j, j, 0).wait()
    idx_load(j, j, 1).wait()
    gath(j, j).start()

  def step(j, carry):
    @pl.when(j + 3 < NCH)
    def _():
      s3 = lax.rem(j + 3, IR)
      idx_load(j + 3, s3, 0).start()
      idx_load(j + 3, s3, 1).start()

    s = lax.rem(j, IR)
    b = lax.rem(j, NBUF)
    gath(s, b).wait()
    pltpu.async_copy(
        rows_v.at[b, pl.ds(0, CH)], acc_sh.at[idxc_r.at[s]], sem_s, add=True
    )

    @pl.when(j >= 1)
    def _():
      scat(lax.rem(j - 1, IR), lax.rem(j - 1, NBUF)).wait()

    @pl.when(j + 2 < NCH)
    def _():
      s2 = lax.rem(j + 2, IR)
      idx_load(j + 2, s2, 0).wait()
      idx_load(j + 2, s2, 1).wait()
      gath(s2, lax.rem(j + 2, NBUF)).start()

    return carry

  lax.fori_loop(0, NCH, step, 0)
  scat(lax.rem(NCH - 1, IR), lax.rem(NCH - 1, NBUF)).wait()
  pltpu.sync_copy(row_hbm.at[pl.ds(base + NCH * CH, TAIL)], idxr_t)
  pltpu.sync_copy(col_hbm.at[pl.ds(base + NCH * CH, TAIL)], idxc_t)
  pltpu.async_copy(
      y_hbm.at[idxr_t], rows_v.at[0, pl.ds(0, TAIL)], sem_g
  ).wait()
  pltpu.sync_copy(rows_v.at[0, pl.ds(0, TAIL)], acc_sh.at[idxc_t], add=True)
  plsc.subcore_barrier()
  out_sl = pl.ds(rbase, RPT)
  pltpu.sync_copy(acc_sh.at[out_sl], p_hbm.at[cid, out_sl])


_mp_kernel = pl.kernel(
    _mp_body,
    out_type=jax.ShapeDtypeStruct((NC, N3, D), jnp.float32),
    mesh=_mesh(),
    scratch_types=[
        pltpu.VMEM((IR, CH), jnp.int32),
        pltpu.VMEM((IR, CH), jnp.int32),
        pltpu.VMEM((TAIL,), jnp.int32),
        pltpu.VMEM((TAIL,), jnp.int32),
        pltpu.VMEM((NBUF, CH, D), jnp.float32),
        pltpu.SemaphoreType.DMA,
        pltpu.SemaphoreType.DMA,
        pltpu.SemaphoreType.DMA,
        pltpu.VMEM_SHARED((N3, D), jnp.float32),
    ],
)


def _fin_body(p_ref, y_ref, dis_ref, out_ref):
  out_ref[...] = dis_ref[...] * (p_ref[0] + p_ref[1] - y_ref[...])


def _finalize(p, y, dis):
  fb = 1000
  return pl.pallas_call(
      _fin_body,
      grid=(N // fb,),
      in_specs=[
          pl.BlockSpec((NC, fb, D), lambda i: (0, i, 0)),
          pl.BlockSpec((fb, D), lambda i: (i, 0)),
          pl.BlockSpec((fb, 1), lambda i: (i, 0)),
      ],
      out_specs=pl.BlockSpec((fb, D), lambda i: (i, 0)),
      out_shape=jax.ShapeDtypeStruct((N, D), jnp.float32),
  )(p, y, dis)


def kernel(x, edge_index):
  col = edge_index[1]
  row = lax.optimization_barrier((edge_index[0], col))[0]
  x_pad = jnp.pad(x, ((0, N3 - N), (0, 0)))
  degp = _deg_kernel(col)
  degc = degp.reshape(NC, N3).T
  y, dis = _norm(x_pad, degc)
  p = _mp_kernel(row, col, y)
  return _finalize(p, y, dis)


# speedup vs baseline: 1.3461x; 1.0002x over previous
import jax
import jax.numpy as jnp
from jax import lax
from jax.experimental import pallas as pl
from jax.experimental.pallas import tpu as pltpu
from jax.experimental.pallas import tpu_sc as plsc

N = 10000
E = 320000
D = 128

NC = 2
NS = 16
EPT = E // (NC * NS)
CH = 104
NCH = 96
TAIL = EPT - CH * NCH
IR = 8
NBUF = 3
N3 = 10240
RPT = N3 // NS


def _mesh():
  return plsc.VectorSubcoreMesh(
      core_axis_name="c", subcore_axis_name="s", num_cores=NC, num_subcores=NS
  )


def _deg_body(col_hbm, degp_hbm, idx_r, idx_t, ones_v, zero_v, sem_i, sem_s,
              deg_sh):
  cid = lax.axis_index("c")
  sid = lax.axis_index("s")
  for i in range(8):
    ones_v[pl.ds(i * 16, 16)] = jnp.ones((16,), jnp.float32)
  for i in range(RPT // 16):
    zero_v[pl.ds(i * 16, 16)] = jnp.zeros((16,), jnp.float32)

  base = pl.multiple_of((cid * NS + sid) * EPT, 8)

  def idx_load(j, slot):
    return pltpu.make_async_copy(
        col_hbm.at[pl.ds(pl.multiple_of(base + j * CH, 8), CH)],
        idx_r.at[slot],
        sem_i,
    )

  def scat(slot):
    return pltpu.make_async_copy(
        ones_v.at[pl.ds(0, CH)], deg_sh.at[idx_r.at[slot]], sem_s
    )

  pltpu.sync_copy(zero_v, deg_sh.at[pl.ds(sid * RPT, RPT)])
  for j in range(3):
    idx_load(j, j).start()
  plsc.subcore_barrier()

  def step(j, carry):
    @pl.when(j + 3 < NCH)
    def _():
      idx_load(j + 3, lax.rem(j + 3, IR)).start()

    s = lax.rem(j, IR)
    idx_load(j, s).wait()
    pltpu.async_copy(
        ones_v.at[pl.ds(0, CH)], deg_sh.at[idx_r.at[s]], sem_s, add=True
    )

    @pl.when(j >= 2)
    def _():
      scat(lax.rem(j - 2, IR)).wait()

    return carry

  lax.fori_loop(0, NCH, step, 0)
  scat(lax.rem(NCH - 2, IR)).wait()
  scat(lax.rem(NCH - 1, IR)).wait()
  pltpu.sync_copy(col_hbm.at[pl.ds(base + NCH * CH, TAIL)], idx_t)
  pltpu.sync_copy(ones_v.at[pl.ds(0, TAIL)], deg_sh.at[idx_t], add=True)
  plsc.subcore_barrier()
  pltpu.sync_copy(
      deg_sh.at[pl.ds(sid * RPT, RPT)],
      degp_hbm.at[pl.ds(cid * N3 + sid * RPT, RPT)],
  )


_deg_kernel = pl.kernel(
    _deg_body,
    out_type=jax.ShapeDtypeStruct((NC * N3,), jnp.float32),
    mesh=_mesh(),
    scratch_types=[
        pltpu.VMEM((IR, CH), jnp.int32),
        pltpu.VMEM((TAIL,), jnp.int32),
        pltpu.VMEM((128,), jnp.float32),
        pltpu.VMEM((RPT,), jnp.float32),
        pltpu.SemaphoreType.DMA,
        pltpu.SemaphoreType.DMA,
        pltpu.VMEM_SHARED((N3,), jnp.float32),
    ],
)


def _norm_body(x_ref, degc_ref, y_ref, dis_ref):
  deg = degc_ref[:, 0:1] + degc_ref[:, 1:2] + 1.0
  dis = lax.rsqrt(deg)
  dis_ref[...] = dis
  y_ref[...] = x_ref[...] * dis


def _norm(x, degc):
  return pl.pallas_call(
      _norm_body,
      out_shape=(
          jax.ShapeDtypeStruct((N3, D), jnp.float32),
          jax.ShapeDtypeStruct((N3, 1), jnp.float32),
      ),
  )(x, degc)


def _mp_body(
    row_hbm, col_hbm, y_hbm, p_hbm, idxr_r, idxc_r, idxr_t, idxc_t, rows_v,
    sem_i, sem_g, sem_s, acc_sh
):
  cid = lax.axis_index("c")
  sid = lax.axis_index("s")
  base = pl.multiple_of((cid * NS + sid) * EPT, 8)
  rbase = pl.multiple_of(sid * RPT, 8)

  def idx_load(j, slot, which):
    src = row_hbm if which == 0 else col_hbm
    dst = idxr_r if which == 0 else idxc_r
    return pltpu.make_async_copy(
        src.at[pl.ds(pl.multiple_of(base + j * CH, 8), CH)],
        dst.at[slot],
        sem_i,
    )

  def gath(slot, b):
    return pltpu.make_async_copy(
        y_hbm.at[idxr_r.at[slot]], rows_v.at[b, pl.ds(0, CH)], sem_g
    )

  def scat(slot, b):
    return pltpu.make_async_copy(
        rows_v.at[b, pl.ds(0, CH)], acc_sh.at[idxc_r.at[slot]], sem_s
    )

  init_sl = pl.ds(rbase, RPT)
  pltpu.sync_copy(y_hbm.at[init_sl], acc_sh.at[init_sl])
  for j in range(3):
    idx_load(j, j, 0).start()
    idx_load(j, j, 1).start()
  plsc.subcore_barrier()
  for j in range(2):
    idx_load(j, j, 0).wait()
    idx_load(j, j, 1).wait()
    gath(j, j).start()

  def step(j, carry):
    @pl.when(j + 3 < NCH)
    def _():
      s3 = lax.rem(j + 3, IR)
      idx_load(j + 3, s3, 0).start()
      idx_load(j + 3, s3, 1).start()

    s = lax.rem(j, IR)
    b = lax.rem(j, NBUF)
    gath(s, b).wait()
    pltpu.async_copy(
        rows_v.at[b, pl.ds(0, CH)], acc_sh.at[idxc_r.at[s]], sem_s, add=True
    )

    @pl.when(j >= 1)
    def _():
      scat(lax.rem(j - 1, IR), lax.rem(j - 1, NBUF)).wait()

    @pl.when(j + 2 < NCH)
    def _():
      s2 = lax.rem(j + 2, IR)
      idx_load(j + 2, s2, 0).wait()
      idx_load(j + 2, s2, 1).wait()
      gath(s2, lax.rem(j + 2, NBUF)).start()

    return carry

  lax.fori_loop(0, NCH, step, 0)
  scat(lax.rem(NCH - 1, IR), lax.rem(NCH - 1, NBUF)).wait()
  pltpu.sync_copy(row_hbm.at[pl.ds(base + NCH * CH, TAIL)], idxr_t)
  pltpu.sync_copy(col_hbm.at[pl.ds(base + NCH * CH, TAIL)], idxc_t)
  pltpu.async_copy(
      y_hbm.at[idxr_t], rows_v.at[0, pl.ds(0, TAIL)], sem_g
  ).wait()
  pltpu.sync_copy(rows_v.at[0, pl.ds(0, TAIL)], acc_sh.at[idxc_t], add=True)
  plsc.subcore_barrier()
  out_sl = pl.ds(rbase, RPT)
  pltpu.sync_copy(acc_sh.at[out_sl], p_hbm.at[cid, out_sl])


_mp_kernel = pl.kernel(
    _mp_body,
    out_type=jax.ShapeDtypeStruct((NC, N3, D), jnp.float32),
    mesh=_mesh(),
    scratch_types=[
        pltpu.VMEM((IR, CH), jnp.int32),
        pltpu.VMEM((IR, CH), jnp.int32),
        pltpu.VMEM((TAIL,), jnp.int32),
        pltpu.VMEM((TAIL,), jnp.int32),
        pltpu.VMEM((NBUF, CH, D), jnp.float32),
        pltpu.SemaphoreType.DMA,
        pltpu.SemaphoreType.DMA,
        pltpu.SemaphoreType.DMA,
        pltpu.VMEM_SHARED((N3, D), jnp.float32),
    ],
)


def _fin_body(p_ref, y_ref, dis_ref, out_ref):
  out_ref[...] = dis_ref[...] * (p_ref[0] + p_ref[1] - y_ref[...])


def _finalize(p, y, dis):
  fb = 2000
  return pl.pallas_call(
      _fin_body,
      grid=(N // fb,),
      in_specs=[
          pl.BlockSpec((NC, fb, D), lambda i: (0, i, 0)),
          pl.BlockSpec((fb, D), lambda i: (i, 0)),
          pl.BlockSpec((fb, 1), lambda i: (i, 0)),
      ],
      out_specs=pl.BlockSpec((fb, D), lambda i: (i, 0)),
      out_shape=jax.ShapeDtypeStruct((N, D), jnp.float32),
  )(p, y, dis)


def kernel(x, edge_index):
  col = edge_index[1]
  colb = lax.optimization_barrier(col)
  row = edge_index[0] + jnp.minimum(colb[0], 0)
  x_pad = jnp.pad(x, ((0, N3 - N), (0, 0)))
  degp = _deg_kernel(col)
  degc = degp.reshape(NC, N3).T
  y, dis = _norm(x_pad, degc)
  p = _mp_kernel(row, col, y)
  return _finalize(p, y, dis)
